# baseline (speedup 1.0000x reference)
import functools

import jax
import jax.numpy as jnp
from jax import lax
from jax.experimental import pallas as pl
from jax.experimental.pallas import tpu as pltpu
from jax.experimental.pallas import tpu_sc as plsc

N = 6890
NW = 32
B = 224
CH = 112
PAD = NW * B
NVEC = B // 16
NCH = B // CH

_mesh = plsc.VectorSubcoreMesh(core_axis_name="c", subcore_axis_name="s")


@functools.partial(
    pl.kernel,
    mesh=_mesh,
    out_type=jax.ShapeDtypeStruct((NW, 16), jnp.float32),
    scratch_types=[
        pltpu.VMEM((B,), jnp.int32),
        pltpu.VMEM((B,), jnp.int32),
        pltpu.VMEM((B,), jnp.int32),
        pltpu.VMEM((B,), jnp.float32),
        pltpu.VMEM((16,), jnp.float32),
        pltpu.SemaphoreType.DMA,
    ],
)
def _gather_partials(p2p_hbm, dist_hbm, sc_hbm, tc_hbm, out_hbm,
                     idx_v, map_v, flat_v, vals_v, acc_v, sem):
    wid = lax.axis_index("s") * 2 + lax.axis_index("c")
    base = wid * B

    pltpu.sync_copy(tc_hbm.at[pl.ds(base, B)], idx_v)
    copies = [
        pltpu.async_copy(p2p_hbm.at[idx_v.at[pl.ds(h * CH, CH)]],
                         map_v.at[pl.ds(h * CH, CH)], sem)
        for h in range(NCH)
    ]
    pltpu.sync_copy(sc_hbm.at[pl.ds(base, B)], idx_v)
    for c in copies:
        c.wait()

    for i in range(NVEC):
        sl = pl.ds(i * 16, 16)
        flat_v[sl] = map_v[sl] * N + idx_v[sl]

    copies = [
        pltpu.async_copy(dist_hbm.at[flat_v.at[pl.ds(h * CH, CH)]],
                         vals_v.at[pl.ds(h * CH, CH)], sem)
        for h in range(NCH)
    ]
    for c in copies:
        c.wait()

    lane = lax.iota(jnp.int32, 16)
    acc = jnp.zeros((16,), jnp.float32)
    for i in range(NVEC):
        g = lane + (base + i * 16)
        acc = acc + jnp.where(g < N, vals_v[pl.ds(i * 16, 16)], 0.0)

    acc_v[...] = acc
    pltpu.sync_copy(acc_v, out_hbm.at[wid])


def _mean_body(x_ref, o_ref):
    o_ref[...] = jnp.sum(x_ref[...], keepdims=True).reshape(1, 1) * (1.0 / N)


_mean_call = pl.pallas_call(
    _mean_body,
    out_shape=jax.ShapeDtypeStruct((1, 1), jnp.float32),
)


def kernel(p2p21, source_distances, source_corr, target_corr):
    p2p = p2p21.astype(jnp.int32)
    tc = jnp.pad(target_corr.astype(jnp.int32), (0, PAD - N))
    sc = jnp.pad(source_corr.astype(jnp.int32), (0, PAD - N))
    dist = source_distances.reshape(-1)
    partials = _gather_partials(p2p, dist, sc, tc)
    return _mean_call(partials)[0, 0]
